# baseline (speedup 1.0000x reference)
import functools

import jax
import jax.numpy as jnp
from jax import lax
from jax.experimental import pallas as pl
from jax.experimental.pallas import tpu as pltpu
from jax.experimental.pallas import tpu_sc as plsc

NF = 26
D = 16
B = 16384
NROWS = B * NF
NC, NS, L = 2, 16, 16
NW = NC * NS
W = NROWS // NW
GI = 128
CHUNK = 1664
NG = CHUNK // GI
NCHUNK = W // CHUNK
KTILE = CHUNK // NF


def _body(x_hbm, table_hbm, bias_hbm, offs_hbm, out_hbm,
          x_v, idx_v, offs_v, bias_t, gbuf0, gbuf1, gsem):
    wid = lax.axis_index("s") * NC + lax.axis_index("c")
    base = wid * W

    pltpu.sync_copy(x_hbm.at[pl.ds(base, W)], x_v)
    pltpu.sync_copy(offs_hbm, offs_v)
    for r in range(KTILE):
        pltpu.sync_copy(bias_hbm, bias_t.at[pl.ds(r * NF, NF)])

    iota = lax.iota(jnp.int32, L)

    def idx_step(r, _):
        for j in range(8):
            p0 = r * 128 + j * L
            fid = jnp.remainder(p0 + iota, NF)
            offs = plsc.load_gather(offs_v, [fid])
            xv = x_v[pl.ds(p0, L)]
            idx_v[pl.ds(p0, L)] = (xv + offs).astype(jnp.int32)
        return _
    lax.fori_loop(0, W // 128, idx_step, 0)

    bufs = (gbuf0, gbuf1)

    def fire(c, buf):
        cps = []
        for g in range(NG):
            cps.append(pltpu.async_copy(
                table_hbm.at[idx_v.at[pl.ds(c * CHUNK + g * GI, GI)]],
                buf.at[pl.ds(g * GI, GI)], gsem))
        return cps

    def drain_add_store(c, buf, cps):
        for cp in cps:
            cp.wait()

        def add_step(r, _):
            for j in range(L):
                row = r * L + j
                buf[row, :] = buf[row, :] + bias_t[row, :]
            return _
        lax.fori_loop(0, CHUNK // L, add_step, 0)
        pltpu.sync_copy(buf, out_hbm.at[pl.ds(base + c * CHUNK, CHUNK)])

    cps = fire(0, bufs[0])
    for c in range(NCHUNK):
        nxt = fire(c + 1, bufs[(c + 1) % 2]) if c + 1 < NCHUNK else None
        drain_add_store(c, bufs[c % 2], cps)
        cps = nxt


@jax.jit
def kernel(x, table, bias, offsets):
    x_flat = x.reshape(NROWS)
    offs_pad = jnp.zeros((32,), jnp.float32).at[:NF].set(offsets)
    mesh = plsc.VectorSubcoreMesh(core_axis_name="c", subcore_axis_name="s",
                                  num_cores=NC, num_subcores=NS)
    out = pl.kernel(
        _body,
        out_type=jax.ShapeDtypeStruct((NROWS, D), jnp.float32),
        mesh=mesh,
        compiler_params=pltpu.CompilerParams(needs_layout_passes=False,
                                             use_tc_tiling_on_sc=False),
        scratch_types=[
            pltpu.VMEM((W,), jnp.float32),
            pltpu.VMEM((W,), jnp.int32),
            pltpu.VMEM((32,), jnp.float32),
            pltpu.VMEM((CHUNK, D), jnp.float32),
            pltpu.VMEM((CHUNK, D), jnp.float32),
            pltpu.VMEM((CHUNK, D), jnp.float32),
            pltpu.SemaphoreType.DMA,
        ],
    )(x_flat, table, bias, offs_pad)
    return out.reshape(B, NF, D)


# speedup vs baseline: 1.2749x; 1.2749x over previous
import jax
import jax.numpy as jnp
from jax import lax
from jax.experimental import pallas as pl
from jax.experimental.pallas import tpu as pltpu
from jax.experimental.pallas import tpu_sc as plsc

NF = 26
D = 16
B = 16384
NC, NS, L = 2, 16, 16
NW = NC * NS
BPW = B // NW
GI = 128
NGF = BPW // GI


def _body(x_hbm, table_hbm, bias_hbm, offs_hbm, out_hbm,
          x_all, idx_all, offs_v, bias_v, obuf0, obuf1, gsem, osem):
    wid = lax.axis_index("s") * NC + lax.axis_index("c")
    b0 = wid * BPW

    pltpu.sync_copy(x_hbm.at[:, pl.ds(b0, BPW)], x_all)
    pltpu.sync_copy(bias_hbm, bias_v)
    pltpu.sync_copy(offs_hbm, offs_v)

    for f in range(NF):
        offv = offs_v[f, :]

        def idx_step(k, _, f=f, offv=offv):
            xv = x_all[f, pl.ds(k * L, L)]
            idx_all[f, pl.ds(k * L, L)] = (xv + offv).astype(jnp.int32)
            return _
        lax.fori_loop(0, BPW // L, idx_step, 0)

    bufs = (obuf0, obuf1)

    def fire(f, buf):
        return [pltpu.async_copy(
            table_hbm.at[idx_all.at[f].at[pl.ds(g * GI, GI)]],
            buf.at[pl.ds(g * GI, GI)], gsem) for g in range(NGF)]

    store_cp = [None, None]
    cps = fire(0, bufs[0])
    for f in range(NF):
        bi = f % 2
        nxt = None
        if f + 1 < NF:
            if store_cp[1 - bi] is not None:
                store_cp[1 - bi].wait()
                store_cp[1 - bi] = None
            nxt = fire(f + 1, bufs[1 - bi])
        for cp in cps:
            cp.wait()
        buf = bufs[bi]
        bias_f = bias_v[f, :]

        def add_step(r, _, buf=buf, bias_f=bias_f):
            for j in range(L):
                row = r * L + j
                buf[row, :] = buf[row, :] + bias_f
            return _
        lax.fori_loop(0, BPW // L, add_step, 0)
        store_cp[bi] = pltpu.async_copy(
            buf, out_hbm.at[f].at[pl.ds(b0, BPW)], osem)
        cps = nxt
    for cp in store_cp:
        if cp is not None:
            cp.wait()


@jax.jit
def kernel(x, table, bias, offsets):
    x_t = x.T
    offs_b = jnp.tile(offsets[:, None], (1, D))
    mesh = plsc.VectorSubcoreMesh(core_axis_name="c", subcore_axis_name="s",
                                  num_cores=NC, num_subcores=NS)
    out = pl.kernel(
        _body,
        out_type=jax.ShapeDtypeStruct((NF, B, D), jnp.float32),
        mesh=mesh,
        compiler_params=pltpu.CompilerParams(needs_layout_passes=False,
                                             use_tc_tiling_on_sc=False),
        scratch_types=[
            pltpu.VMEM((NF, BPW), jnp.float32),
            pltpu.VMEM((NF, BPW), jnp.int32),
            pltpu.VMEM((NF, D), jnp.float32),
            pltpu.VMEM((NF, D), jnp.float32),
            pltpu.VMEM((BPW, D), jnp.float32),
            pltpu.VMEM((BPW, D), jnp.float32),
            pltpu.SemaphoreType.DMA,
            pltpu.SemaphoreType.DMA,
        ],
    )(x_t, table, bias, offs_b)
    return out.transpose(1, 0, 2)


# speedup vs baseline: 1.4125x; 1.1079x over previous
import jax
import jax.numpy as jnp
from jax import lax
from jax.experimental import pallas as pl
from jax.experimental.pallas import tpu as pltpu
from jax.experimental.pallas import tpu_sc as plsc

NF = 26
D = 16
B = 16384
NR = 100000 * NF
NC, NS, L = 2, 16, 16
NW = NC * NS
BPW = B // NW
GI = 128
NGF = BPW // GI

NCHT = 5078
NCH = 160
TAIL = NR - NCHT * 512

_mesh = plsc.VectorSubcoreMesh(core_axis_name="c", subcore_axis_name="s",
                               num_cores=NC, num_subcores=NS)


def _k1_body(tt_hbm, tail_hbm, out_hbm, v0, v1, o0, o1, tv,
             g0, g1, s0, s1):
    wid = lax.axis_index("s") * NC + lax.axis_index("c")
    s_ch = (wid * NCHT) // NW
    iota = lax.iota(jnp.int32, L)

    def ci(i):
        return jnp.minimum(s_ch + i, NCHT - 1)

    def fire(i, buf, sem):
        pltpu.async_copy(tt_hbm.at[:, pl.ds(ci(i) * 512, 512)], buf, sem)

    def wait_in(buf, sem):
        pltpu.make_async_copy(tt_hbm.at[:, pl.ds(0, 512)], buf, sem).wait()

    def store(i, obuf, sem):
        pltpu.async_copy(obuf, out_hbm.at[pl.ds(ci(i) * 64, 64)], sem)

    def wait_out(obuf, sem):
        pltpu.make_async_copy(obuf, out_hbm.at[pl.ds(0, 64)], sem).wait()

    def emit(buf, obuf, nrows=512):
        def step(rr, _):
            for j in range(8):
                rloc = rr * 8 + j
                col = plsc.load_gather(buf, [iota, jnp.broadcast_to(rloc, (L,))])
                obuf[rr, pl.ds(j * 16, L)] = col
            return _
        lax.fori_loop(0, nrows // 8, step, 0)

    fire(0, v0, g0)
    fire(1, v1, g1)

    def body(p, _):
        k = p * 2
        wait_in(v0, g0)

        @pl.when(p > 0)
        def _w0():
            wait_out(o0, s0)
        emit(v0, o0)
        store(k, o0, s0)
        fire(k + 2, v0, g0)
        wait_in(v1, g1)

        @pl.when(p > 0)
        def _w1():
            wait_out(o1, s1)
        emit(v1, o1)
        store(k + 1, o1, s1)
        fire(k + 3, v1, g1)
        return _
    lax.fori_loop(0, NCH // 2, body, 0)
    wait_in(v0, g0)
    wait_in(v1, g1)
    wait_out(o0, s0)
    wait_out(o1, s1)
    pltpu.sync_copy(tail_hbm, tv)
    for rr in range(TAIL // 8):
        for j in range(8):
            o0[rr, pl.ds(j * 16, L)] = tv[rr * 8 + j, :]
    pltpu.sync_copy(o0.at[pl.ds(0, TAIL // 8)],
                    out_hbm.at[pl.ds(NCHT * 64, TAIL // 8)])


def _k2_body(x_hbm, table_hbm, bias_hbm, offs_hbm, out_hbm,
             x_all, idx_all, offs_v, bias_v, obuf0, obuf1, gsem, osem):
    wid = lax.axis_index("s") * NC + lax.axis_index("c")
    b0 = wid * BPW

    pltpu.sync_copy(x_hbm.at[:, pl.ds(b0, BPW)], x_all)
    pltpu.sync_copy(bias_hbm, bias_v)
    pltpu.sync_copy(offs_hbm, offs_v)

    for f in range(NF):
        offv = offs_v[f, :]

        def idx_step(k, _, f=f, offv=offv):
            xv = x_all[f, pl.ds(k * L, L)]
            idx_all[f, pl.ds(k * L, L)] = (xv + offv).astype(jnp.int32)
            return _
        lax.fori_loop(0, BPW // L, idx_step, 0)

    bufs = (obuf0, obuf1)

    def fire(f, buf):
        return [pltpu.async_copy(
            table_hbm.at[idx_all.at[f].at[pl.ds(g * GI, GI)]],
            buf.at[pl.ds(g * GI, GI)], gsem) for g in range(NGF)]

    store_cp = [None, None]
    cps = fire(0, bufs[0])
    for f in range(NF):
        bi = f % 2
        nxt = None
        if f + 1 < NF:
            if store_cp[1 - bi] is not None:
                store_cp[1 - bi].wait()
                store_cp[1 - bi] = None
            nxt = fire(f + 1, bufs[1 - bi])
        for cp in cps:
            cp.wait()
        buf = bufs[bi]
        bias_f = bias_v[f, :]

        def add_step(r, _, buf=buf, bias_f=bias_f):
            for j in range(L):
                row = r * L + j
                buf[row, :] = buf[row, :] + bias_f
            return _
        lax.fori_loop(0, BPW // L, add_step, 0)
        store_cp[bi] = pltpu.async_copy(
            buf, out_hbm.at[f].at[pl.ds(b0, BPW)], osem)
        cps = nxt
    for cp in store_cp:
        if cp is not None:
            cp.wait()


@jax.jit
def kernel(x, table, bias, offsets):
    tt = table.T
    k1 = pl.kernel(
        _k1_body,
        out_type=jax.ShapeDtypeStruct((NR // 8, 128), jnp.float32),
        mesh=_mesh,
        compiler_params=pltpu.CompilerParams(needs_layout_passes=False,
                                             use_tc_tiling_on_sc=True),
        scratch_types=[
            pltpu.VMEM((16, 512), jnp.float32),
            pltpu.VMEM((16, 512), jnp.float32),
            pltpu.VMEM((64, 128), jnp.float32),
            pltpu.VMEM((64, 128), jnp.float32),
            pltpu.VMEM((TAIL, D), jnp.float32),
            pltpu.SemaphoreType.DMA,
            pltpu.SemaphoreType.DMA,
            pltpu.SemaphoreType.DMA,
            pltpu.SemaphoreType.DMA,
        ],
    )
    t_lin = k1(tt, table[NCHT * 512:]).reshape(NR, D)

    x_t = x.T
    offs_b = jnp.tile(offsets[:, None], (1, D))
    k2 = pl.kernel(
        _k2_body,
        out_type=jax.ShapeDtypeStruct((NF, B, D), jnp.float32),
        mesh=_mesh,
        compiler_params=pltpu.CompilerParams(needs_layout_passes=False,
                                             use_tc_tiling_on_sc=False),
        scratch_types=[
            pltpu.VMEM((NF, BPW), jnp.float32),
            pltpu.VMEM((NF, BPW), jnp.int32),
            pltpu.VMEM((NF, D), jnp.float32),
            pltpu.VMEM((NF, D), jnp.float32),
            pltpu.VMEM((BPW, D), jnp.float32),
            pltpu.VMEM((BPW, D), jnp.float32),
            pltpu.SemaphoreType.DMA,
            pltpu.SemaphoreType.DMA,
        ],
    )
    out = k2(x_t, t_lin, bias, offs_b)
    return out.transpose(1, 0, 2)


# speedup vs baseline: 2.1721x; 1.5378x over previous
import jax
import jax.numpy as jnp
from jax import lax
from jax.experimental import pallas as pl
from jax.experimental.pallas import tpu as pltpu
from jax.experimental.pallas import tpu_sc as plsc

NF = 26
D = 16
B = 16384
NR = 100000 * NF
NC, NS, L = 2, 16, 16
NW = NC * NS
BPW = B // NW
GI = 128
NGF = BPW // GI

NCHT = 5078
NCH = 160
TAIL = NR - NCHT * 512

_mesh = plsc.VectorSubcoreMesh(core_axis_name="c", subcore_axis_name="s",
                               num_cores=NC, num_subcores=NS)


def _k1_body(tt_hbm, tail_hbm, out_hbm, v0, v1, o0, o1, tv,
             g0, g1, s0, s1):
    wid = lax.axis_index("s") * NC + lax.axis_index("c")
    s_ch = (wid * NCHT) // NW
    iota = lax.iota(jnp.int32, L)

    def ci(i):
        return jnp.minimum(s_ch + i, NCHT - 1)

    def fire(i, buf, sem):
        pltpu.async_copy(tt_hbm.at[:, pl.ds(ci(i) * 512, 512)], buf, sem)

    def wait_in(buf, sem):
        pltpu.make_async_copy(tt_hbm.at[:, pl.ds(0, 512)], buf, sem).wait()

    def store(i, obuf, sem):
        pltpu.async_copy(obuf, out_hbm.at[pl.ds(ci(i) * 64, 64)], sem)

    def wait_out(obuf, sem):
        pltpu.make_async_copy(obuf, out_hbm.at[pl.ds(0, 64)], sem).wait()

    def emit(buf, obuf, nrows=512):
        @plsc.parallel_loop(0, nrows // 8, unroll=4)
        def _e(rr):
            for j in range(8):
                rloc = rr * 8 + j
                col = plsc.load_gather(buf, [iota, jnp.broadcast_to(rloc, (L,))])
                obuf[rr, pl.ds(j * 16, L)] = col

    fire(0, v0, g0)
    fire(1, v1, g1)

    def body(p, _):
        k = p * 2
        wait_in(v0, g0)

        @pl.when(p > 0)
        def _w0():
            wait_out(o0, s0)
        emit(v0, o0)
        store(k, o0, s0)
        fire(k + 2, v0, g0)
        wait_in(v1, g1)

        @pl.when(p > 0)
        def _w1():
            wait_out(o1, s1)
        emit(v1, o1)
        store(k + 1, o1, s1)
        fire(k + 3, v1, g1)
        return _
    lax.fori_loop(0, NCH // 2, body, 0)
    wait_in(v0, g0)
    wait_in(v1, g1)
    wait_out(o0, s0)
    wait_out(o1, s1)
    pltpu.sync_copy(tail_hbm, tv)
    for rr in range(TAIL // 8):
        for j in range(8):
            o0[rr, pl.ds(j * 16, L)] = tv[rr * 8 + j, :]
    pltpu.sync_copy(o0.at[pl.ds(0, TAIL // 8)],
                    out_hbm.at[pl.ds(NCHT * 64, TAIL // 8)])


def _k2_body(x_hbm, table_hbm, bias_hbm, offs_hbm, out_hbm,
             x_all, idx_all, offs_v, bias_v, obuf0, obuf1, gsem, osem):
    wid = lax.axis_index("s") * NC + lax.axis_index("c")
    b0 = wid * BPW

    pltpu.sync_copy(x_hbm.at[:, pl.ds(b0, BPW)], x_all)
    pltpu.sync_copy(bias_hbm, bias_v)
    pltpu.sync_copy(offs_hbm, offs_v)

    for f in range(NF):
        offv = offs_v[f, :]

        def idx_step(k, _, f=f, offv=offv):
            xv = x_all[f, pl.ds(k * L, L)]
            idx_all[f, pl.ds(k * L, L)] = (xv + offv).astype(jnp.int32)
            return _
        lax.fori_loop(0, BPW // L, idx_step, 0)

    bufs = (obuf0, obuf1)

    def fire(f, buf):
        return [pltpu.async_copy(
            table_hbm.at[idx_all.at[f].at[pl.ds(g * GI, GI)]],
            buf.at[pl.ds(g * GI, GI)], gsem) for g in range(NGF)]

    store_cp = [None, None]
    cps = fire(0, bufs[0])
    for f in range(NF):
        bi = f % 2
        nxt = None
        if f + 1 < NF:
            if store_cp[1 - bi] is not None:
                store_cp[1 - bi].wait()
                store_cp[1 - bi] = None
            nxt = fire(f + 1, bufs[1 - bi])
        for cp in cps:
            cp.wait()
        buf = bufs[bi]
        bias_f = bias_v[f, :]

        def add_step(r, _, buf=buf, bias_f=bias_f):
            for j in range(L):
                row = r * L + j
                buf[row, :] = buf[row, :] + bias_f
            return _
        lax.fori_loop(0, BPW // L, add_step, 0)
        store_cp[bi] = pltpu.async_copy(
            buf, out_hbm.at[f].at[pl.ds(b0, BPW)], osem)
        cps = nxt
    for cp in store_cp:
        if cp is not None:
            cp.wait()


@jax.jit
def kernel(x, table, bias, offsets):
    tt = table.T
    k1 = pl.kernel(
        _k1_body,
        out_type=jax.ShapeDtypeStruct((NR // 8, 128), jnp.float32),
        mesh=_mesh,
        compiler_params=pltpu.CompilerParams(needs_layout_passes=False,
                                             use_tc_tiling_on_sc=True),
        scratch_types=[
            pltpu.VMEM((16, 512), jnp.float32),
            pltpu.VMEM((16, 512), jnp.float32),
            pltpu.VMEM((64, 128), jnp.float32),
            pltpu.VMEM((64, 128), jnp.float32),
            pltpu.VMEM((TAIL, D), jnp.float32),
            pltpu.SemaphoreType.DMA,
            pltpu.SemaphoreType.DMA,
            pltpu.SemaphoreType.DMA,
            pltpu.SemaphoreType.DMA,
        ],
    )
    t_lin = k1(tt, table[NCHT * 512:]).reshape(NR, D)

    x_t = x.T
    offs_b = jnp.tile(offsets[:, None], (1, D))
    k2 = pl.kernel(
        _k2_body,
        out_type=jax.ShapeDtypeStruct((NF, B, D), jnp.float32),
        mesh=_mesh,
        compiler_params=pltpu.CompilerParams(needs_layout_passes=False,
                                             use_tc_tiling_on_sc=False),
        scratch_types=[
            pltpu.VMEM((NF, BPW), jnp.float32),
            pltpu.VMEM((NF, BPW), jnp.int32),
            pltpu.VMEM((NF, D), jnp.float32),
            pltpu.VMEM((NF, D), jnp.float32),
            pltpu.VMEM((BPW, D), jnp.float32),
            pltpu.VMEM((BPW, D), jnp.float32),
            pltpu.SemaphoreType.DMA,
            pltpu.SemaphoreType.DMA,
        ],
    )
    out = k2(x_t, t_lin, bias, offs_b)
    return out.transpose(1, 0, 2)


# speedup vs baseline: 2.1723x; 1.0001x over previous
import jax
import jax.numpy as jnp
from jax import lax
from jax.experimental import pallas as pl
from jax.experimental.pallas import tpu as pltpu
from jax.experimental.pallas import tpu_sc as plsc

NF = 26
D = 16
B = 16384
NR = 100000 * NF
NC, NS, L = 2, 16, 16
NW = NC * NS
BPW = B // NW
GI = 128
NGF = BPW // GI

KCH = 1024
NCHT = 2539
NCH = 80
TAIL = NR - NCHT * KCH

_mesh = plsc.VectorSubcoreMesh(core_axis_name="c", subcore_axis_name="s",
                               num_cores=NC, num_subcores=NS)


def _k1_body(tt_hbm, tail_hbm, out_hbm, v0, v1, o0, o1, tv,
             g0, g1, s0, s1):
    wid = lax.axis_index("s") * NC + lax.axis_index("c")
    s_ch = (wid * NCHT) // NW
    iota = lax.iota(jnp.int32, L)

    def ci(i):
        return jnp.minimum(s_ch + i, NCHT - 1)

    def fire(i, buf, sem):
        pltpu.async_copy(tt_hbm.at[:, pl.ds(ci(i) * KCH, KCH)], buf, sem)

    def wait_in(buf, sem):
        pltpu.make_async_copy(tt_hbm.at[:, pl.ds(0, KCH)], buf, sem).wait()

    def store(i, obuf, sem):
        pltpu.async_copy(obuf, out_hbm.at[pl.ds(ci(i) * (KCH // 8), KCH // 8)], sem)

    def wait_out(obuf, sem):
        pltpu.make_async_copy(obuf, out_hbm.at[pl.ds(0, KCH // 8)], sem).wait()

    def emit(buf, obuf, nrows=KCH):
        @plsc.parallel_loop(0, nrows // 8, unroll=4)
        def _e(rr):
            for j in range(8):
                rloc = rr * 8 + j
                col = plsc.load_gather(buf, [iota, jnp.broadcast_to(rloc, (L,))])
                obuf[rr, pl.ds(j * 16, L)] = col

    fire(0, v0, g0)
    fire(1, v1, g1)

    def body(p, _):
        k = p * 2
        wait_in(v0, g0)

        @pl.when(p > 0)
        def _w0():
            wait_out(o0, s0)
        emit(v0, o0)
        store(k, o0, s0)
        fire(k + 2, v0, g0)
        wait_in(v1, g1)

        @pl.when(p > 0)
        def _w1():
            wait_out(o1, s1)
        emit(v1, o1)
        store(k + 1, o1, s1)
        fire(k + 3, v1, g1)
        return _
    lax.fori_loop(0, NCH // 2, body, 0)
    wait_in(v0, g0)
    wait_in(v1, g1)
    wait_out(o0, s0)
    wait_out(o1, s1)
    pltpu.sync_copy(tail_hbm, tv)
    for rr in range(TAIL // 8):
        for j in range(8):
            o0[rr, pl.ds(j * 16, L)] = tv[rr * 8 + j, :]
    pltpu.sync_copy(o0.at[pl.ds(0, TAIL // 8)],
                    out_hbm.at[pl.ds(NCHT * (KCH // 8), TAIL // 8)])


def _k2_body(x_hbm, table_hbm, bias_hbm, offs_hbm, out_hbm,
             x_all, idx_all, offs_v, bias_v, obuf0, obuf1, gsem, osem):
    wid = lax.axis_index("s") * NC + lax.axis_index("c")
    b0 = wid * BPW

    pltpu.sync_copy(x_hbm.at[:, pl.ds(b0, BPW)], x_all)
    pltpu.sync_copy(bias_hbm, bias_v)
    pltpu.sync_copy(offs_hbm, offs_v)

    for f in range(NF):
        offv = offs_v[f, :]

        def idx_step(k, _, f=f, offv=offv):
            xv = x_all[f, pl.ds(k * L, L)]
            idx_all[f, pl.ds(k * L, L)] = (xv + offv).astype(jnp.int32)
            return _
        lax.fori_loop(0, BPW // L, idx_step, 0)

    bufs = (obuf0, obuf1)

    def fire(f, buf):
        return [pltpu.async_copy(
            table_hbm.at[idx_all.at[f].at[pl.ds(g * GI, GI)]],
            buf.at[pl.ds(g * GI, GI)], gsem) for g in range(NGF)]

    store_cp = [None, None]
    cps = fire(0, bufs[0])
    for f in range(NF):
        bi = f % 2
        nxt = None
        if f + 1 < NF:
            if store_cp[1 - bi] is not None:
                store_cp[1 - bi].wait()
                store_cp[1 - bi] = None
            nxt = fire(f + 1, bufs[1 - bi])
        for cp in cps:
            cp.wait()
        buf = bufs[bi]
        bias_f = bias_v[f, :]

        def add_step(r, _, buf=buf, bias_f=bias_f):
            for j in range(L):
                row = r * L + j
                buf[row, :] = buf[row, :] + bias_f
            return _
        lax.fori_loop(0, BPW // L, add_step, 0)
        store_cp[bi] = pltpu.async_copy(
            buf, out_hbm.at[f].at[pl.ds(b0, BPW)], osem)
        cps = nxt
    for cp in store_cp:
        if cp is not None:
            cp.wait()


@jax.jit
def kernel(x, table, bias, offsets):
    tt = table.T
    k1 = pl.kernel(
        _k1_body,
        out_type=jax.ShapeDtypeStruct((NR // 8, 128), jnp.float32),
        mesh=_mesh,
        compiler_params=pltpu.CompilerParams(needs_layout_passes=False,
                                             use_tc_tiling_on_sc=True),
        scratch_types=[
            pltpu.VMEM((16, KCH), jnp.float32),
            pltpu.VMEM((16, KCH), jnp.float32),
            pltpu.VMEM((KCH // 8, 128), jnp.float32),
            pltpu.VMEM((KCH // 8, 128), jnp.float32),
            pltpu.VMEM((TAIL, D), jnp.float32),
            pltpu.SemaphoreType.DMA,
            pltpu.SemaphoreType.DMA,
            pltpu.SemaphoreType.DMA,
            pltpu.SemaphoreType.DMA,
        ],
    )
    t_lin = k1(tt, table[NCHT * KCH:]).reshape(NR, D)

    x_t = x.T
    offs_b = jnp.tile(offsets[:, None], (1, D))
    k2 = pl.kernel(
        _k2_body,
        out_type=jax.ShapeDtypeStruct((NF, B, D), jnp.float32),
        mesh=_mesh,
        compiler_params=pltpu.CompilerParams(needs_layout_passes=False,
                                             use_tc_tiling_on_sc=False),
        scratch_types=[
            pltpu.VMEM((NF, BPW), jnp.float32),
            pltpu.VMEM((NF, BPW), jnp.int32),
            pltpu.VMEM((NF, D), jnp.float32),
            pltpu.VMEM((NF, D), jnp.float32),
            pltpu.VMEM((BPW, D), jnp.float32),
            pltpu.VMEM((BPW, D), jnp.float32),
            pltpu.SemaphoreType.DMA,
            pltpu.SemaphoreType.DMA,
        ],
    )
    out = k2(x_t, t_lin, bias, offs_b)
    return out.transpose(1, 0, 2)
